# Initial kernel scaffold; baseline (speedup 1.0000x reference)
import jax
import jax.numpy as jnp
from jax.experimental import pallas as pl


def kernel(h, edge_index, W_edge, b_edge, W_node, b_node):
    raise NotImplementedError("write your pallas kernel here")


# speedup vs baseline: 26.3295x; 26.3295x over previous
import functools

import jax
import jax.numpy as jnp
from jax import lax
from jax.experimental import pallas as pl
from jax.experimental.pallas import tpu as pltpu
from jax.experimental.pallas import tpu_sc as plsc

N = 10000
H = 128
E = 320000
NW = 32
LANES = 16
ROWS = 80
EPW = ROWS * 128
EPAD = NW * EPW
NP = 10240
TBL = 10240


def _tc_scores_body(w_ref, h_ref, b_ref, o_ref):
    acc = jax.lax.dot_general(
        w_ref[...], h_ref[...], (((1,), (1,)), ((), ())),
        preferred_element_type=jnp.float32)
    o_ref[...] = acc + b_ref[:, :1]


def _tc_softmax_body(p0_ref, p1_ref, sb_ref, o_ref):
    x = p0_ref[...] + p1_ref[...] + sb_ref[...]
    ridx = lax.broadcasted_iota(jnp.int32, x.shape, 0)
    lidx = lax.broadcasted_iota(jnp.int32, x.shape, 1)
    valid = ridx * 128 + lidx < N
    x = jnp.where(valid, x, -jnp.inf)
    m = jnp.max(x)
    e = jnp.exp(x - m)
    s = jnp.sum(e)
    o_ref[...] = e * (1.0 / s)


def _sc_edge_body(a_hbm, c_hbm, srcr_hbm, dstr_hbm, z_hbm, out_hbm,
                  a_v, c_v, src_v, dst_v, vals_v, acc_sh, sem):
    cid = lax.axis_index("c")
    sid = lax.axis_index("s")
    wid = sid * 2 + cid

    pltpu.sync_copy(a_hbm, a_v.at[pl.ds(0, N)])
    pltpu.sync_copy(c_hbm, c_v.at[pl.ds(0, N)])
    pltpu.sync_copy(srcr_hbm.at[wid], src_v)
    pltpu.sync_copy(dstr_hbm.at[wid], dst_v)

    @pl.when(sid == 0)
    def _init():
        pltpu.sync_copy(z_hbm, acc_sh)

    def row(r, carry):
        for l in range(8):
            si = src_v[r, pl.ds(l * LANES, LANES)]
            di = dst_v[r, pl.ds(l * LANES, LANES)]
            va = plsc.load_gather(a_v, [si])
            vc = plsc.load_gather(c_v, [di])
            vals_v[r, pl.ds(l * LANES, LANES)] = va + vc
        return carry

    lax.fori_loop(0, ROWS, row, 0)

    plsc.subcore_barrier()

    for r0 in range(0, ROWS, 8):
        cps = [pltpu.async_copy(vals_v.at[r], acc_sh.at[dst_v.at[r]], sem,
                                add=True)
               for r in range(r0, r0 + 8)]
        for cp in cps:
            cp.wait()

    plsc.subcore_barrier()

    @pl.when(sid == 0)
    def _flush():
        pltpu.sync_copy(acc_sh, out_hbm.at[cid])


@functools.cache
def _sc_edge():
    return pl.kernel(
        _sc_edge_body,
        out_type=jax.ShapeDtypeStruct((2, NP), jnp.float32),
        mesh=plsc.VectorSubcoreMesh(core_axis_name="c", subcore_axis_name="s"),
        compiler_params=pltpu.CompilerParams(needs_layout_passes=False),
        scratch_types=[
            pltpu.VMEM((TBL,), jnp.float32),
            pltpu.VMEM((TBL,), jnp.float32),
            pltpu.VMEM((ROWS, 128), jnp.int32),
            pltpu.VMEM((ROWS, 128), jnp.int32),
            pltpu.VMEM((ROWS, 128), jnp.float32),
            pltpu.MemorySpace.VMEM_SHARED((NP,), jnp.float32),
            pltpu.SemaphoreType.DMA,
        ],
    )


@jax.jit
def kernel(h, edge_index, W_edge, b_edge, W_node, b_node):
    h = h.astype(jnp.float32)
    ei = edge_index.astype(jnp.int32)
    pad = jnp.full((EPAD - E,), N, dtype=jnp.int32)
    srcr = jnp.concatenate([ei[0], pad]).reshape(NW, ROWS, 128)
    dstr = jnp.concatenate([ei[1], pad]).reshape(NW, ROWS, 128)

    w3 = jnp.zeros((8, H), jnp.float32)
    w3 = w3.at[0].set(W_edge[:H]).at[1].set(W_edge[H:]).at[2].set(W_node)
    bias = jnp.zeros((8, 1), jnp.float32)
    bias = bias.at[1, 0].set(b_edge).at[2, 0].set(b_node)
    bias = jnp.broadcast_to(bias, (8, 128))

    scores = pl.pallas_call(
        _tc_scores_body,
        out_shape=jax.ShapeDtypeStruct((8, N), jnp.float32),
    )(w3, h, bias)

    a = scores[0]
    c = scores[1]
    sb = jnp.pad(scores[2], (0, NP - N)).reshape(ROWS, 128)
    zeros = jnp.zeros((NP,), jnp.float32)

    parts = _sc_edge()(a, c, srcr, dstr, zeros)

    p0 = parts[0].reshape(ROWS, 128)
    p1 = parts[1].reshape(ROWS, 128)

    out = pl.pallas_call(
        _tc_softmax_body,
        out_shape=jax.ShapeDtypeStruct((ROWS, 128), jnp.float32),
    )(p0, p1, sb)

    return out.reshape(NP)[:N]
